# Initial kernel scaffold; baseline (speedup 1.0000x reference)
import jax
import jax.numpy as jnp
from jax.experimental import pallas as pl


def kernel(user_emb, item_emb, edge_index, edge_weight):
    raise NotImplementedError("write your pallas kernel here")


# speedup vs baseline: 5.3196x; 5.3196x over previous
import functools

import jax
import jax.numpy as jnp
from jax import lax
from jax.experimental import pallas as pl
from jax.experimental.pallas import tpu as pltpu
from jax.experimental.pallas import tpu_sc as plsc

NUM_USERS = 25000
NUM_ITEMS = 25000
N = NUM_USERS + NUM_ITEMS
E = 800000
D = 64
N_LAYERS = 3

NC = 2
NS = 16
HALF = N // NC
HALF_PAD = 25088
TRASH = HALF
RPT = HALF_PAD // NS
ZROWS = 32

CH = 128
GROUP = 8
EROWS = -(-E // (CH * GROUP)) * GROUP
EP = EROWS * CH
NOCT = EROWS // GROUP
SUB = 2


def _layer_body(table, src2, dst2, w2, out, acc, src_v, dst_v, w_v, rows_v,
                zeros_v, gsem):
    c = lax.axis_index("c")
    s = lax.axis_index("s")

    def zfill(i, _):
        r = i // 4
        col = (i % 4) * 16
        zeros_v[r, pl.ds(col, 16)] = jnp.zeros((16,), jnp.float32)
        return _

    lax.fori_loop(0, ZROWS * 4, zfill, None)
    lo_acc = s * RPT
    for z in range(RPT // ZROWS):
        pltpu.sync_copy(zeros_v, acc.at[pl.ds(lo_acc + z * ZROWS, ZROWS), :])
    plsc.subcore_barrier()

    lo_o = (s * NOCT) // NS
    hi_o = ((s + 1) * NOCT) // NS
    base = c * HALF
    nrows = GROUP

    def process(o, _):
        r0 = o * GROUP
        pltpu.sync_copy(src2.at[pl.ds(r0, nrows)], src_v)
        pltpu.sync_copy(dst2.at[pl.ds(r0, nrows)], dst_v)
        pltpu.sync_copy(w2.at[pl.ds(r0, nrows)], w_v)

        def remap(t, _):
            for j in range(nrows):
                v = dst_v[j, pl.ds(t * 16, 16)] - base
                ok = (v >= 0) & (v < HALF)
                dst_v[j, pl.ds(t * 16, 16)] = jnp.where(ok, v, TRASH)
            return _

        lax.fori_loop(0, CH // 16, remap, None)

        for sub in range(nrows // SUB):
            handles = [
                pltpu.async_copy(table.at[src_v.at[sub * SUB + j]],
                                 rows_v.at[pl.ds(j * CH, CH)], gsem)
                for j in range(SUB)
            ]
            for h in handles:
                h.wait()

            def scale(t, _):
                e0 = t * 16
                for j in range(SUB):
                    wvec = w_v[sub * SUB + j, pl.ds(e0, 16)]
                    for k in range(16):
                        wv = wvec[k]
                        row = j * CH + e0 + k
                        for dblk in range(D // 16):
                            sl = pl.ds(dblk * 16, 16)
                            rows_v[row, sl] = rows_v[row, sl] * wv
                return _

            lax.fori_loop(0, CH // 16, scale, None)

            for j in range(SUB):
                pltpu.sync_copy(rows_v.at[pl.ds(j * CH, CH)],
                                acc.at[dst_v.at[sub * SUB + j]], add=True)
        return _

    lax.fori_loop(lo_o, hi_o, process, None)

    plsc.subcore_barrier()

    lo = s * RPT

    @pl.when(s < NS - 1)
    def _():
        pltpu.sync_copy(acc.at[pl.ds(lo, RPT), :],
                        out.at[pl.ds(base + lo, RPT), :])

    @pl.when(s == NS - 1)
    def _():
        last = HALF - (NS - 1) * RPT
        pltpu.sync_copy(acc.at[pl.ds(lo, last), :],
                        out.at[pl.ds(base + lo, last), :])


@functools.partial(
    pl.kernel,
    out_type=jax.ShapeDtypeStruct((N, D), jnp.float32),
    mesh=plsc.VectorSubcoreMesh(core_axis_name="c", subcore_axis_name="s"),
    compiler_params=pltpu.CompilerParams(use_tc_tiling_on_sc=False),
    scratch_types=[
        pltpu.VMEM_SHARED((HALF_PAD, D), jnp.float32),
        pltpu.VMEM((GROUP, CH), jnp.int32),
        pltpu.VMEM((GROUP, CH), jnp.int32),
        pltpu.VMEM((GROUP, CH), jnp.float32),
        pltpu.VMEM((SUB * CH, D), jnp.float32),
        pltpu.VMEM((ZROWS, D), jnp.float32),
        pltpu.SemaphoreType.DMA,
    ],
)
def _propagate(table, src2, dst2, w2, out, acc, src_v, dst_v, w_v, rows_v,
               zeros_v, gsem):
    _layer_body(table, src2, dst2, w2, out, acc, src_v, dst_v, w_v, rows_v,
                zeros_v, gsem)


def _mean_body(a_ref, b_ref, c_ref, d_ref, o_ref):
    o_ref[...] = (a_ref[...] + b_ref[...] + c_ref[...] + d_ref[...]) * 0.25


_MEAN_BLOCK = 2000


def _mean4(t0, t1, t2, t3):
    spec = pl.BlockSpec((_MEAN_BLOCK, D), lambda i: (i, 0))
    return pl.pallas_call(
        _mean_body,
        grid=(N // _MEAN_BLOCK,),
        in_specs=[spec, spec, spec, spec],
        out_specs=spec,
        out_shape=jax.ShapeDtypeStruct((N, D), jnp.float32),
    )(t0, t1, t2, t3)


def kernel(user_emb, item_emb, edge_index, edge_weight):
    all_emb = jnp.concatenate([user_emb, item_emb], axis=0)
    pad = EP - E
    src2 = jnp.pad(edge_index[0], (0, pad)).reshape(EROWS, CH)
    dst2 = jnp.pad(edge_index[1], (0, pad)).reshape(EROWS, CH)
    w2 = jnp.pad(edge_weight, (0, pad)).reshape(EROWS, CH)

    tables = [all_emb]
    for _ in range(N_LAYERS):
        tables.append(_propagate(tables[-1], src2, dst2, w2))

    light_out = _mean4(*tables)
    return light_out[:NUM_USERS], light_out[NUM_USERS:]


# speedup vs baseline: 5.9739x; 1.1230x over previous
import functools

import jax
import jax.numpy as jnp
from jax import lax
from jax.experimental import pallas as pl
from jax.experimental.pallas import tpu as pltpu
from jax.experimental.pallas import tpu_sc as plsc

NUM_USERS = 25000
NUM_ITEMS = 25000
N = NUM_USERS + NUM_ITEMS
E = 800000
D = 64
N_LAYERS = 3

NC = 2
NS = 16
HALF = N // NC
HALF_PAD = 25024
TRASH = HALF
RPT = HALF_PAD // NS

CH = 128
GROUP = 8
EROWS = -(-E // (CH * GROUP)) * GROUP
EP = EROWS * CH
NOCT = EROWS // GROUP
NBUF = 3


def _layer_body(table, src2, dst2, w2, out, acc, src_v, dst_v, w_v, rows_v,
                gsem, ssem, lsem):
    c = lax.axis_index("c")
    s = lax.axis_index("s")
    base = c * HALF

    def zfill(i, _):
        r = i // 4
        col = (i % 4) * 16
        rows_v[0, r, pl.ds(col, 16)] = jnp.zeros((16,), jnp.float32)
        return _

    lax.fori_loop(0, CH * 4, zfill, None)
    lo_acc = s * RPT
    for z in range(RPT // CH):
        pltpu.sync_copy(rows_v.at[0],
                        acc.at[pl.ds(lo_acc + z * CH, CH), :])
    zrem = RPT % CH
    if zrem:
        pltpu.sync_copy(rows_v.at[0, pl.ds(0, zrem)],
                        acc.at[pl.ds(lo_acc + (RPT // CH) * CH, zrem), :])
    plsc.subcore_barrier()

    lo_o = (s * NOCT) // NS
    hi_o = ((s + 1) * NOCT) // NS
    jlo = lo_o * GROUP
    jhi = hi_o * GROUP

    def lin_fire(o, p):
        r0 = o * GROUP
        return [
            pltpu.async_copy(src2.at[pl.ds(r0, GROUP)], src_v.at[p], lsem),
            pltpu.async_copy(dst2.at[pl.ds(r0, GROUP)], dst_v.at[p], lsem),
            pltpu.async_copy(w2.at[pl.ds(r0, GROUP)], w_v.at[p], lsem),
        ]

    def lin_drain(p):
        pltpu.make_async_copy(src2.at[pl.ds(0, GROUP)], src_v.at[p],
                              lsem).wait()
        pltpu.make_async_copy(dst2.at[pl.ds(0, GROUP)], dst_v.at[p],
                              lsem).wait()
        pltpu.make_async_copy(w2.at[pl.ds(0, GROUP)], w_v.at[p], lsem).wait()

    def gather_fire(j):
        o = j // GROUP
        r = j - o * GROUP
        p = o & 1
        b = j % 3
        pltpu.async_copy(table.at[src_v.at[p, r]], rows_v.at[b], gsem)

    def gather_drain():
        pltpu.make_async_copy(table.at[pl.ds(0, CH)], rows_v.at[0],
                              gsem).wait()

    def scatter_fire(j):
        o = j // GROUP
        r = j - o * GROUP
        p = o & 1
        b = j % 3
        pltpu.async_copy(rows_v.at[b], acc.at[dst_v.at[p, r]], ssem,
                         add=True)

    def scatter_drain():
        pltpu.make_async_copy(rows_v.at[0], acc.at[pl.ds(0, CH), :],
                              ssem).wait()

    p0 = lo_o & 1
    pltpu.sync_copy(src2.at[pl.ds(jlo, GROUP)], src_v.at[p0])
    pltpu.sync_copy(dst2.at[pl.ds(jlo, GROUP)], dst_v.at[p0])
    pltpu.sync_copy(w2.at[pl.ds(jlo, GROUP)], w_v.at[p0])
    gather_fire(jlo)
    gather_fire(jlo + 1)

    def job(j, _):
        o = j // GROUP
        r = j - o * GROUP
        p = o & 1
        b = j % 3

        @pl.when(r == 0)
        def _():

            def remap(t, _):
                for jj in range(GROUP):
                    v = dst_v[p, jj, pl.ds(t * 16, 16)] - base
                    ok = (v >= 0) & (v < HALF)
                    dst_v[p, jj, pl.ds(t * 16, 16)] = jnp.where(ok, v, TRASH)
                return _

            lax.fori_loop(0, CH // 16, remap, None)

        @pl.when(r == GROUP - 2)
        def _():
            lin_drain(1 - p)

        gather_drain()

        def scale(t, _):
            e0 = t * 16
            wvec = w_v[p, r, pl.ds(e0, 16)]
            for k in range(16):
                wv = wvec[k]
                for dblk in range(D // 16):
                    sl = pl.ds(dblk * 16, 16)
                    rows_v[b, e0 + k, sl] = rows_v[b, e0 + k, sl] * wv
            return _

        lax.fori_loop(0, CH // 16, scale, None)

        scatter_fire(j)

        @pl.when(j > jlo)
        def _():
            scatter_drain()

        gather_fire(j + 2)

        @pl.when(r == 0)
        def _():
            lin_fire(jnp.minimum(o + 1, NOCT - 1), 1 - p)
        return _

    lax.fori_loop(jlo, jhi, job, None)

    gather_drain()
    gather_drain()
    scatter_drain()

    plsc.subcore_barrier()

    lo = s * RPT

    @pl.when(s < NS - 1)
    def _():
        pltpu.sync_copy(acc.at[pl.ds(lo, RPT), :],
                        out.at[pl.ds(base + lo, RPT), :])

    @pl.when(s == NS - 1)
    def _():
        last = HALF - (NS - 1) * RPT
        pltpu.sync_copy(acc.at[pl.ds(lo, last), :],
                        out.at[pl.ds(base + lo, last), :])


@functools.partial(
    pl.kernel,
    out_type=jax.ShapeDtypeStruct((N, D), jnp.float32),
    mesh=plsc.VectorSubcoreMesh(core_axis_name="c", subcore_axis_name="s"),
    compiler_params=pltpu.CompilerParams(use_tc_tiling_on_sc=False),
    scratch_types=[
        pltpu.VMEM_SHARED((HALF_PAD, D), jnp.float32),
        pltpu.VMEM((2, GROUP, CH), jnp.int32),
        pltpu.VMEM((2, GROUP, CH), jnp.int32),
        pltpu.VMEM((2, GROUP, CH), jnp.float32),
        pltpu.VMEM((NBUF, CH, D), jnp.float32),
        pltpu.SemaphoreType.DMA,
        pltpu.SemaphoreType.DMA,
        pltpu.SemaphoreType.DMA,
    ],
)
def _propagate(table, src2, dst2, w2, out, acc, src_v, dst_v, w_v, rows_v,
               gsem, ssem, lsem):
    _layer_body(table, src2, dst2, w2, out, acc, src_v, dst_v, w_v, rows_v,
                gsem, ssem, lsem)


def _mean_body(a_ref, b_ref, c_ref, d_ref, o_ref):
    o_ref[...] = (a_ref[...] + b_ref[...] + c_ref[...] + d_ref[...]) * 0.25


_MEAN_BLOCK = 2000


def _mean4(t0, t1, t2, t3):
    spec = pl.BlockSpec((_MEAN_BLOCK, D), lambda i: (i, 0))
    return pl.pallas_call(
        _mean_body,
        grid=(N // _MEAN_BLOCK,),
        in_specs=[spec, spec, spec, spec],
        out_specs=spec,
        out_shape=jax.ShapeDtypeStruct((N, D), jnp.float32),
    )(t0, t1, t2, t3)


def kernel(user_emb, item_emb, edge_index, edge_weight):
    all_emb = jnp.concatenate([user_emb, item_emb], axis=0)
    pad = EP - E
    src2 = jnp.pad(edge_index[0], (0, pad)).reshape(EROWS, CH)
    dst2 = jnp.pad(edge_index[1], (0, pad)).reshape(EROWS, CH)
    w2 = jnp.pad(edge_weight, (0, pad)).reshape(EROWS, CH)

    tables = [all_emb]
    for _ in range(N_LAYERS):
        tables.append(_propagate(tables[-1], src2, dst2, w2))

    light_out = _mean4(*tables)
    return light_out[:NUM_USERS], light_out[NUM_USERS:]
